# baseline (speedup 1.0000x reference)
import jax
import jax.numpy as jnp
from jax.experimental import pallas as pl
from jax.experimental.pallas import tpu as pltpu

def _body(x_hbm, o_hbm, sem):
    pltpu.make_async_copy(x_hbm, o_hbm, sem).start()
    pltpu.make_async_copy(x_hbm, o_hbm, sem).wait()

def kernel(input):
    x = input.reshape(8192, 4096)
    out = pl.pallas_call(
        _body,
        out_shape=jax.ShapeDtypeStruct((8192, 4096), jnp.float32),
        in_specs=[pl.BlockSpec(memory_space=pl.ANY)],
        out_specs=pl.BlockSpec(memory_space=pl.ANY),
        scratch_shapes=[pltpu.SemaphoreType.DMA],
    )(x)
    return out.reshape(input.shape)


# speedup vs baseline: 7.2436x; 7.2436x over previous
import functools

import jax
import jax.numpy as jnp
from jax import lax
from jax.experimental import pallas as pl
from jax.experimental.pallas import tpu as pltpu
from jax.experimental.pallas import tpu_sc as plsc

_N = 2 * 4096 * 4096
_NW = 32
_PER_W = _N // _NW
_CHUNK = 16384
_NCHUNK = _PER_W // _CHUNK
_NVEC = _CHUNK // 16

_A = 1.5957691216057308
_B = 0.07135481283247087


def _sc_gelu_body(x_hbm, o_hbm, inb, outb, insem, outsem):
    c = lax.axis_index("c")
    s = lax.axis_index("s")
    wid = s * 2 + c
    base = wid * _PER_W

    def in_copy(k, slot):
        return pltpu.make_async_copy(
            x_hbm.at[pl.ds(base + k * _CHUNK, _CHUNK)], inb.at[slot], insem.at[slot]
        )

    def out_copy(k, slot):
        return pltpu.make_async_copy(
            outb.at[slot], o_hbm.at[pl.ds(base + k * _CHUNK, _CHUNK)], outsem.at[slot]
        )

    in_copy(0, 0).start()
    in_copy(1, 1).start()

    def chunk_body(k, _):
        slot = lax.rem(k, 2)
        in_copy(k, slot).wait()

        @pl.when(k >= 2)
        def _():
            out_copy(k - 2, slot).wait()

        def vec(j, _):
            sl = pl.ds(j * 16, 16)
            x = inb[slot, sl]
            u2 = jnp.minimum(_A * x + _B * (x * x * x), 30.0)
            t = jnp.exp(u2)
            outb[slot, sl] = x * t / (t + 1.0)
            return 0

        lax.fori_loop(0, _NVEC, vec, 0)
        out_copy(k, slot).start()

        @pl.when(k + 2 < _NCHUNK)
        def _():
            in_copy(k + 2, slot).start()

        return 0

    lax.fori_loop(0, _NCHUNK, chunk_body, 0)
    out_copy(_NCHUNK - 2, 0).wait()
    out_copy(_NCHUNK - 1, 1).wait()


@functools.cache
def _build_sc_gelu():
    mesh = plsc.VectorSubcoreMesh(core_axis_name="c", subcore_axis_name="s")
    return pl.kernel(
        _sc_gelu_body,
        out_type=jax.ShapeDtypeStruct((_N,), jnp.float32),
        mesh=mesh,
        scratch_types=[
            pltpu.VMEM((2, _CHUNK), jnp.float32),
            pltpu.VMEM((2, _CHUNK), jnp.float32),
            pltpu.SemaphoreType.DMA((2,)),
            pltpu.SemaphoreType.DMA((2,)),
        ],
    )


def kernel(input):
    out = _build_sc_gelu()(input.reshape(_N))
    return out.reshape(input.shape)


# speedup vs baseline: 48.9682x; 6.7602x over previous
import jax
import jax.numpy as jnp
from jax.experimental import pallas as pl
from jax.experimental.pallas import tpu as pltpu

_ROWS = 8192
_COLS = 4096
_MAXCHUNK = 512
_NBUF = 3

_SCH = [64, 64, 128, 256] + [512] * 14 + [256, 128, 64, 64]
assert sum(_SCH) == _ROWS
_OFFS = [sum(_SCH[:i]) for i in range(len(_SCH))]
_NCH = len(_SCH)


def _gelu(x):
    return 0.5 * x * (1.0 + jax.lax.erf(x * 0.7071067811865476))


def _body(x_hbm, o_hbm, inbuf, outbuf, insem, outsem):
    def in_copy(i):
        slot = i % _NBUF
        return pltpu.make_async_copy(
            x_hbm.at[pl.ds(_OFFS[i], _SCH[i]), :],
            inbuf.at[slot, pl.ds(0, _SCH[i]), :],
            insem.at[slot],
        )

    def out_copy(i):
        slot = i % _NBUF
        return pltpu.make_async_copy(
            outbuf.at[slot, pl.ds(0, _SCH[i]), :],
            o_hbm.at[pl.ds(_OFFS[i], _SCH[i]), :],
            outsem.at[slot],
        )

    for b in range(_NBUF):
        in_copy(b).start()

    for i in range(_NCH):
        slot = i % _NBUF
        in_copy(i).wait()
        if i >= _NBUF:
            out_copy(i - _NBUF).wait()
        n = _SCH[i]
        outbuf[slot, :n, :] = _gelu(inbuf[slot, :n, :])
        out_copy(i).start()
        if i + _NBUF < _NCH:
            in_copy(i + _NBUF).start()

    for i in range(_NCH - _NBUF, _NCH):
        out_copy(i).wait()


def kernel(input):
    x = input.reshape(_ROWS, _COLS)
    out = pl.pallas_call(
        _body,
        out_shape=jax.ShapeDtypeStruct((_ROWS, _COLS), jnp.float32),
        in_specs=[pl.BlockSpec(memory_space=pl.ANY)],
        out_specs=pl.BlockSpec(memory_space=pl.ANY),
        scratch_shapes=[
            pltpu.VMEM((_NBUF, _MAXCHUNK, _COLS), jnp.float32),
            pltpu.VMEM((_NBUF, _MAXCHUNK, _COLS), jnp.float32),
            pltpu.SemaphoreType.DMA((_NBUF,)),
            pltpu.SemaphoreType.DMA((_NBUF,)),
        ],
    )(x)
    return out.reshape(input.shape)


# speedup vs baseline: 49.3576x; 1.0080x over previous
import jax
import jax.numpy as jnp
from jax.experimental import pallas as pl
from jax.experimental.pallas import tpu as pltpu

_ROWS = 8192
_COLS = 4096
_MAXCHUNK = 512
_NBUF = 3

_SCH = [128, 128, 256] + [512] * 14 + [256, 128, 128]
assert sum(_SCH) == _ROWS
_OFFS = [sum(_SCH[:i]) for i in range(len(_SCH))]
_NCH = len(_SCH)


def _gelu(x):
    return 0.5 * x * (1.0 + jax.lax.erf(x * 0.7071067811865476))


def _body(x_hbm, o_hbm, inbuf, outbuf, insem, outsem):
    def in_copy(i):
        slot = i % _NBUF
        return pltpu.make_async_copy(
            x_hbm.at[pl.ds(_OFFS[i], _SCH[i]), :],
            inbuf.at[slot, pl.ds(0, _SCH[i]), :],
            insem.at[slot],
        )

    def out_copy(i):
        slot = i % _NBUF
        return pltpu.make_async_copy(
            outbuf.at[slot, pl.ds(0, _SCH[i]), :],
            o_hbm.at[pl.ds(_OFFS[i], _SCH[i]), :],
            outsem.at[slot],
        )

    for b in range(_NBUF):
        in_copy(b).start()

    for i in range(_NCH):
        slot = i % _NBUF
        in_copy(i).wait()
        if i >= _NBUF:
            out_copy(i - _NBUF).wait()
        n = _SCH[i]
        outbuf[slot, :n, :] = _gelu(inbuf[slot, :n, :])
        out_copy(i).start()
        if i + _NBUF < _NCH:
            in_copy(i + _NBUF).start()

    for i in range(_NCH - _NBUF, _NCH):
        out_copy(i).wait()


def kernel(input):
    x = input.reshape(_ROWS, _COLS)
    out = pl.pallas_call(
        _body,
        out_shape=jax.ShapeDtypeStruct((_ROWS, _COLS), jnp.float32),
        in_specs=[pl.BlockSpec(memory_space=pl.ANY)],
        out_specs=pl.BlockSpec(memory_space=pl.ANY),
        scratch_shapes=[
            pltpu.VMEM((_NBUF, _MAXCHUNK, _COLS), jnp.float32),
            pltpu.VMEM((_NBUF, _MAXCHUNK, _COLS), jnp.float32),
            pltpu.SemaphoreType.DMA((_NBUF,)),
            pltpu.SemaphoreType.DMA((_NBUF,)),
        ],
    )(x)
    return out.reshape(input.shape)
